# Initial kernel scaffold; baseline (speedup 1.0000x reference)
import jax
import jax.numpy as jnp
from jax.experimental import pallas as pl


def kernel(rel_bias):
    raise NotImplementedError("write your pallas kernel here")


# speedup vs baseline: 41.5927x; 41.5927x over previous
import jax
import jax.numpy as jnp
from jax import lax
from jax.experimental import pallas as pl
from jax.experimental.pallas import tpu as pltpu
from jax.experimental.pallas import tpu_sc as plsc

L = 2048
H = 16
T = 2 * L - 1
CP = 4096


def _body(copies_hbm, out_hbm, copies_v, sem):
    nc = 2
    c = lax.axis_index("c")
    s = lax.axis_index("s")
    wid = s * nc + c
    h = wid // nc
    half = wid % nc
    i0 = half * (L // 2)

    pltpu.sync_copy(
        copies_hbm.at[pl.ds(pl.multiple_of(h * 8 * CP, 8), 8 * CP)], copies_v
    )

    def emit(b, _):
        cps = []
        for t in range(8):
            i = i0 + b * 8 + t
            start = (L - 1) - i
            sh = start & 7
            src_off = pl.multiple_of(sh * CP + (start - sh), 8)
            dst_off = pl.multiple_of((h * L + i) * L, 8)
            cps.append(
                pltpu.async_copy(
                    copies_v.at[pl.ds(src_off, L)],
                    out_hbm.at[pl.ds(dst_off, L)],
                    sem,
                )
            )
        for cp in cps:
            cp.wait()
        return _

    lax.fori_loop(0, (L // 2) // 8, emit, None)


@jax.jit
def _run(rel_bias):
    rb_rev = jnp.pad(rel_bias[::-1], ((0, 8), (0, 0)))
    shifted = jnp.stack([rb_rev[sh : sh + CP] for sh in range(8)], 0)
    copies = jnp.transpose(shifted, (2, 0, 1)).reshape(-1)
    k = pl.kernel(
        _body,
        mesh=plsc.VectorSubcoreMesh(core_axis_name="c", subcore_axis_name="s"),
        out_type=jax.ShapeDtypeStruct((H * L * L,), jnp.float32),
        scratch_types=[
            pltpu.VMEM((8 * CP,), jnp.float32),
            pltpu.SemaphoreType.DMA,
        ],
    )
    return k(copies).reshape(1, H, L, L)


def kernel(rel_bias):
    return _run(rel_bias)


# speedup vs baseline: 41.7437x; 1.0036x over previous
import jax
import jax.numpy as jnp
from jax import lax
from jax.experimental import pallas as pl
from jax.experimental.pallas import tpu as pltpu
from jax.experimental.pallas import tpu_sc as plsc

L = 2048
H = 16
T = 2 * L - 1
CP = 4096


def _body(copies_hbm, out_hbm, copies_v, sem):
    nc = 2
    c = lax.axis_index("c")
    s = lax.axis_index("s")
    wid = s * nc + c
    h = wid // nc
    half = wid % nc
    i0 = half * (L // 2)

    pltpu.sync_copy(
        copies_hbm.at[pl.ds(pl.multiple_of(h * 8 * CP, 8), 8 * CP)], copies_v
    )

    K = 8

    def fire(b):
        for t in range(K):
            i = i0 + b * K + t
            start = (L - 1) - i
            sh = start & 7
            src_off = pl.multiple_of(sh * CP + (start - sh), 8)
            dst_off = pl.multiple_of((h * L + i) * L, 8)
            pltpu.async_copy(
                copies_v.at[pl.ds(src_off, L)],
                out_hbm.at[pl.ds(dst_off, L)],
                sem,
            )

    def drain_batch():
        for _t in range(K):
            pltpu.make_async_copy(
                copies_v.at[pl.ds(0, L)], out_hbm.at[pl.ds(0, L)], sem
            ).wait()

    nb = (L // 2) // K
    fire(0)

    def emit(b, _):
        fire(b + 1)
        drain_batch()
        return _

    lax.fori_loop(0, nb - 1, emit, None)
    drain_batch()


@jax.jit
def _run(rel_bias):
    rb_rev = jnp.pad(rel_bias[::-1], ((0, 8), (0, 0)))
    shifted = jnp.stack([rb_rev[sh : sh + CP] for sh in range(8)], 0)
    copies = jnp.transpose(shifted, (2, 0, 1)).reshape(-1)
    k = pl.kernel(
        _body,
        mesh=plsc.VectorSubcoreMesh(core_axis_name="c", subcore_axis_name="s"),
        out_type=jax.ShapeDtypeStruct((H * L * L,), jnp.float32),
        scratch_types=[
            pltpu.VMEM((8 * CP,), jnp.float32),
            pltpu.SemaphoreType.DMA,
        ],
    )
    return k(copies).reshape(1, H, L, L)


def kernel(rel_bias):
    return _run(rel_bias)
